# baseline (speedup 1.0000x reference)
import functools
import jax
import jax.numpy as jnp
from jax import lax
from jax.experimental import pallas as pl
from jax.experimental.pallas import tpu as pltpu
from jax.experimental.pallas import tpu_sc as plsc

_UPER = 100
_ALPHA = 1.0
_GAMMA = 0.5
_MIN_KEEP = 1

_B_SC = 4096
_NW = 32
_G = 16
_GD = 64


def _rows_kernel(x_ref, tgt_ref, v_ref, acc_ref):
    i = pl.program_id(0)
    x = x_ref[...]
    tgt = tgt_ref[...]
    R, C = x.shape
    tgt_col = tgt[:, None]
    m = jnp.max(x, axis=1, keepdims=True)
    e = jnp.exp(x - m)
    s = jax.lax.dot_general(e, jnp.ones((C, 1), jnp.float32),
                            (((1,), (0,)), ((), ())),
                            preferred_element_type=jnp.float32)
    col = jax.lax.broadcasted_iota(jnp.int32, (R, C), 1)
    colf = col.astype(jnp.float32)
    predrev = jnp.max(jnp.where(x == m, C - colf, 0.0), axis=1, keepdims=True)
    xt = jnp.max(jnp.where(col == tgt_col, x, -jnp.inf), axis=1, keepdims=True)
    l = m + jnp.log(s) - xt
    pred_f = C - predrev
    w = _ALPHA * jnp.sqrt(jnp.abs(pred_f - tgt_col.astype(jnp.float32)))
    num_mask = tgt_col < _UPER
    v = jnp.where(num_mask, -1.0, l)
    v_ref[...] = v[:, 0]
    sum_l = jnp.sum(l)
    numer = jnp.sum(jnp.where(num_mask, (1.0 + w) * l, 0.0))
    n_num = jnp.sum(num_mask.astype(jnp.float32))
    lane = jax.lax.broadcasted_iota(jnp.int32, (1, 128), 1)
    part = (jnp.where(lane == 0, sum_l, 0.0)
            + jnp.where(lane == 1, numer, 0.0)
            + jnp.where(lane == 2, n_num, 0.0))

    @pl.when(i == 0)
    def _():
        acc_ref[...] = jnp.zeros_like(acc_ref)

    acc_ref[...] += part


def _sc_rows_body(x_hbm, tgt_hbm, m_hbm, s_hbm, xt_hbm, pred_hbm,
                  buf, tgt_v, m_v, s_v, xt_v, pred_v):
    B, C = x_hbm.shape
    rows_w = _B_SC // _NW
    base_row = B - _B_SC
    wid = lax.axis_index("s") * 2 + lax.axis_index("c")
    my_base = wid * rows_w
    lanes = lax.iota(jnp.int32, 16)

    _dn = lax.GatherDimensionNumbers(offset_dims=(), collapsed_slice_dims=(0,),
                                     start_index_map=(0,))

    def _rot(vec, k):
        idx = jnp.bitwise_and(lanes + k, 15)
        return lax.gather(vec, idx[:, None], _dn, (1,),
                          mode=lax.GatherScatterMode.PROMISE_IN_BOUNDS)

    def _allreduce(vec, op):
        for k in (8, 4, 2, 1):
            vec = op(vec, _rot(vec, k))
        return vec
    tail_ok = lanes >= 8
    all_ok = lanes >= 0
    neg_inf = jnp.full((16,), -jnp.inf, jnp.float32)
    n_chunks = (C + 15) // 16
    last_start = C - 16

    pltpu.sync_copy(tgt_hbm.at[pl.ds(base_row + my_base, rows_w)], tgt_v)

    def group_body(gd, _):
        pltpu.sync_copy(
            x_hbm.at[pl.ds(base_row + my_base + gd * _GD, _GD), :], buf)

        def row_body(r, carry):
            mvec, svec, pvec, xtvec = carry
            row = sg * _G + r
            t_bcast = lax.gather(
                tgt16, (jnp.zeros((16,), jnp.int32) + r)[:, None],
                lax.GatherDimensionNumbers(offset_dims=(),
                                           collapsed_slice_dims=(0,),
                                           start_index_map=(0,)),
                (1,), mode=lax.GatherScatterMode.PROMISE_IN_BOUNDS)

            lanemax = neg_inf
            argstart = jnp.zeros((16,), jnp.int32)
            xtacc = neg_inf
            for c in range(n_chunks):
                start = last_start if c == n_chunks - 1 else c * 16
                chunk = buf[row, pl.ds(start, 16)]
                if c == n_chunks - 1:
                    chunk = jnp.where(tail_ok, chunk, neg_inf)
                better = chunk > lanemax
                lanemax = jnp.where(better, chunk, lanemax)
                argstart = jnp.where(
                    better, jnp.full((16,), start, jnp.int32), argstart)
                hit = (start + lanes) == t_bcast
                xtacc = jnp.where(hit, chunk, xtacc)
            m_r = _allreduce(lanemax, jnp.maximum)
            xt_r = _allreduce(xtacc, jnp.maximum)
            idxs = argstart + lanes
            pred_r = _allreduce(
                jnp.where(lanemax == m_r, idxs, jnp.int32(2 ** 30)),
                jnp.minimum)

            esum = jnp.zeros((16,), jnp.float32)
            for c in range(n_chunks):
                start = last_start if c == n_chunks - 1 else c * 16
                chunk = buf[row, pl.ds(start, 16)]
                e = jnp.exp(chunk - m_r)
                if c == n_chunks - 1:
                    e = jnp.where(tail_ok, e, jnp.zeros((16,), jnp.float32))
                esum = esum + e
            s_r = _allreduce(esum, jnp.add)

            sel = lanes == r
            mvec = jnp.where(sel, m_r, mvec)
            svec = jnp.where(sel, s_r, svec)
            pvec = jnp.where(sel, pred_r, pvec)
            xtvec = jnp.where(sel, xt_r, xtvec)
            return (mvec, svec, pvec, xtvec)

        zf = jnp.zeros((16,), jnp.float32)
        for sg in range(_GD // _G):
            tgt16 = tgt_v[pl.ds(gd * _GD + sg * _G, 16)]
            mvec, svec, pvec, xtvec = lax.fori_loop(
                0, _G, row_body, (zf, zf, jnp.zeros((16,), jnp.int32), zf))
            m_v[pl.ds(gd * _GD + sg * _G, 16)] = mvec
            s_v[pl.ds(gd * _GD + sg * _G, 16)] = svec
            xt_v[pl.ds(gd * _GD + sg * _G, 16)] = xtvec
            pred_v[pl.ds(gd * _GD + sg * _G, 16)] = pvec
        return 0

    lax.fori_loop(0, rows_w // _GD, group_body, 0)

    pltpu.sync_copy(m_v, m_hbm.at[pl.ds(my_base, rows_w)])
    pltpu.sync_copy(s_v, s_hbm.at[pl.ds(my_base, rows_w)])
    pltpu.sync_copy(xt_v, xt_hbm.at[pl.ds(my_base, rows_w)])
    pltpu.sync_copy(pred_v, pred_hbm.at[pl.ds(my_base, rows_w)])


def _sc_rows(inputs, targets):
    rows_w = _B_SC // _NW
    f32 = jnp.float32
    return pl.kernel(
        _sc_rows_body,
        mesh=plsc.VectorSubcoreMesh(core_axis_name="c", subcore_axis_name="s"),
        out_type=[
            jax.ShapeDtypeStruct((_B_SC,), f32),
            jax.ShapeDtypeStruct((_B_SC,), f32),
            jax.ShapeDtypeStruct((_B_SC,), f32),
            jax.ShapeDtypeStruct((_B_SC,), jnp.int32),
        ],
        scratch_types=[
            pltpu.VMEM((_GD, 1000), f32),
            pltpu.VMEM((rows_w,), jnp.int32),
            pltpu.VMEM((rows_w,), f32),
            pltpu.VMEM((rows_w,), f32),
            pltpu.VMEM((rows_w,), f32),
            pltpu.VMEM((rows_w,), jnp.int32),
        ],
    )(inputs, targets)


def _final_kernel(v_ref, acc_ref, ms_ref, ss_ref, xts_ref, preds_ref,
                  tgts_ref, out_ref):
    v_tc = v_ref[...]
    sum_l = acc_ref[0, 0]
    numer_sum = acc_ref[0, 1]
    n_num = acc_ref[0, 2]

    m_sc = ms_ref[...]
    s_sc = ss_ref[...]
    xt_sc = xts_ref[...]
    pred_sc = preds_ref[...]
    tgt_sc = tgts_ref[...]
    l_sc = m_sc + jnp.log(s_sc) - xt_sc
    w_sc = _ALPHA * jnp.sqrt(jnp.abs(pred_sc - tgt_sc).astype(jnp.float32))
    num_mask_sc = tgt_sc < _UPER
    v_sc = jnp.where(num_mask_sc, -1.0, l_sc)
    sum_l = sum_l + jnp.sum(l_sc)
    numer_sum = numer_sum + jnp.sum(
        jnp.where(num_mask_sc, (1.0 + w_sc) * l_sc, 0.0))
    n_num = n_num + jnp.sum(num_mask_sc.astype(jnp.float32))

    v = jnp.concatenate([v_tc, v_sc])
    bf = jnp.float32(v.shape[0])
    celoss = sum_l / bf
    n_cls_i = jnp.int32(v.shape[0]) - n_num.astype(jnp.int32)
    cls_sum_all = jnp.sum(jnp.where(v >= 0.0, v, 0.0))
    k = (7 * n_cls_i) // 10

    def body(_, lohi):
        lo, hi = lohi
        mid = lo + (hi - lo + 1) // 2
        t = jax.lax.bitcast_convert_type(mid, jnp.float32)
        ge = jnp.sum((v >= t).astype(jnp.int32)) >= k
        return (jnp.where(ge, mid, lo), jnp.where(ge, hi, mid - 1))

    lo, _ = jax.lax.fori_loop(0, 32, body, (jnp.int32(0), jnp.int32(0x7F800000)))
    t = jax.lax.bitcast_convert_type(lo, jnp.float32)
    gt = v > t
    cnt_gt = jnp.sum(gt.astype(jnp.int32))
    sum_gt = jnp.sum(jnp.where(gt, v, 0.0))
    topk_sum = sum_gt + (k - cnt_gt).astype(jnp.float32) * t

    use_topk = k >= _MIN_KEEP
    cls_sum = jnp.where(n_cls_i > 0, jnp.where(use_topk, topk_sum, cls_sum_all), 0.0)
    valid_num = jnp.where(use_topk, k.astype(jnp.float32), float(_MIN_KEEP))
    cls_size = jnp.where(n_cls_i > 0, valid_num, 0.0)
    numerical_loss = (cls_sum + numer_sum) / (n_num + cls_size + 1e-9)
    out_ref[0] = numerical_loss
    out_ref[1] = celoss / numerical_loss
    out_ref[2] = cls_sum / (cls_size + 1e-9)
    out_ref[3] = numer_sum / (n_num + 1e-9)
    out_ref[4] = cls_size
    out_ref[5] = n_cls_i.astype(jnp.float32)
    out_ref[6] = n_num


def kernel(inputs, targets):
    B, C = inputs.shape
    B_TC = B - _B_SC
    R = 1024

    m_sc, s_sc, xt_sc, pred_sc = _sc_rows(inputs, targets)

    v_tc, acc = pl.pallas_call(
        _rows_kernel,
        grid=(B_TC // R,),
        in_specs=[
            pl.BlockSpec((R, C), lambda i: (i, 0)),
            pl.BlockSpec((R,), lambda i: (i,)),
        ],
        out_specs=[
            pl.BlockSpec((R,), lambda i: (i,)),
            pl.BlockSpec((1, 128), lambda i: (0, 0)),
        ],
        out_shape=[
            jax.ShapeDtypeStruct((B_TC,), jnp.float32),
            jax.ShapeDtypeStruct((1, 128), jnp.float32),
        ],
    )(inputs, targets)

    out = pl.pallas_call(
        _final_kernel,
        out_specs=pl.BlockSpec(memory_space=pltpu.SMEM),
        out_shape=jax.ShapeDtypeStruct((8,), jnp.float32),
    )(v_tc, acc, m_sc, s_sc, xt_sc, pred_sc, targets[B_TC:])

    return (out[0], out[1], out[2], out[3], out[4], out[5], out[6])


# speedup vs baseline: 1.4246x; 1.4246x over previous
import jax
import jax.numpy as jnp
from jax.experimental import pallas as pl
from jax.experimental.pallas import tpu as pltpu

_UPER = 100
_ALPHA = 1.0
_GAMMA = 0.5
_MIN_KEEP = 1


def _fused_kernel(x_ref, tgt_ref, out_ref, v_s, acc_s):
    i = pl.program_id(0)
    nb = pl.num_programs(0)
    x = x_ref[...]
    tgt = tgt_ref[...]
    R, C = x.shape
    tgt_col = tgt[:, None]
    m = jnp.max(x, axis=1, keepdims=True)
    e = jnp.exp(x - m)
    s = jax.lax.dot_general(e, jnp.ones((C, 1), jnp.float32),
                            (((1,), (0,)), ((), ())),
                            preferred_element_type=jnp.float32)
    col = jax.lax.broadcasted_iota(jnp.int32, (R, C), 1)
    colf = col.astype(jnp.float32)
    predrev = jnp.max(jnp.where(x == m, C - colf, 0.0), axis=1, keepdims=True)
    xt = jnp.max(jnp.where(col == tgt_col, x, -jnp.inf), axis=1, keepdims=True)
    l = m + jnp.log(s) - xt
    pred_f = C - predrev
    w = _ALPHA * jnp.sqrt(jnp.abs(pred_f - tgt_col.astype(jnp.float32)))
    num_mask = tgt_col < _UPER
    vblk = jnp.where(num_mask, -1.0, l)
    v_s[pl.ds(i * R, R)] = vblk[:, 0]
    sum_l = jnp.sum(l)
    numer = jnp.sum(jnp.where(num_mask, (1.0 + w) * l, 0.0))
    n_num_p = jnp.sum(num_mask.astype(jnp.float32))
    lane = jax.lax.broadcasted_iota(jnp.int32, (1, 128), 1)
    part = (jnp.where(lane == 0, sum_l, 0.0)
            + jnp.where(lane == 1, numer, 0.0)
            + jnp.where(lane == 2, n_num_p, 0.0))

    @pl.when(i == 0)
    def _():
        acc_s[...] = jnp.zeros_like(acc_s)

    acc_s[...] += part

    @pl.when(i == nb - 1)
    def _():
        v = v_s[...]
        bf = jnp.float32(v.shape[0])
        sum_l_t = acc_s[0, 0]
        numer_sum = acc_s[0, 1]
        n_num = acc_s[0, 2]
        celoss = sum_l_t / bf
        n_cls_i = jnp.int32(v.shape[0]) - n_num.astype(jnp.int32)
        cls_sum_all = jnp.sum(jnp.where(v >= 0.0, v, 0.0))
        k = (7 * n_cls_i) // 10

        def body(_, lohi):
            lo, hi = lohi
            mid = lo + (hi - lo + 1) // 2
            t = jax.lax.bitcast_convert_type(mid, jnp.float32)
            ge = jnp.sum((v >= t).astype(jnp.int32)) >= k
            return (jnp.where(ge, mid, lo), jnp.where(ge, hi, mid - 1))

        lo, _ = jax.lax.fori_loop(0, 32, body,
                                  (jnp.int32(0), jnp.int32(0x7F800000)))
        t = jax.lax.bitcast_convert_type(lo, jnp.float32)
        gt = v > t
        cnt_gt = jnp.sum(gt.astype(jnp.int32))
        sum_gt = jnp.sum(jnp.where(gt, v, 0.0))
        topk_sum = sum_gt + (k - cnt_gt).astype(jnp.float32) * t

        use_topk = k >= _MIN_KEEP
        cls_sum = jnp.where(n_cls_i > 0,
                            jnp.where(use_topk, topk_sum, cls_sum_all), 0.0)
        valid_num = jnp.where(use_topk, k.astype(jnp.float32), float(_MIN_KEEP))
        cls_size = jnp.where(n_cls_i > 0, valid_num, 0.0)
        numerical_loss = (cls_sum + numer_sum) / (n_num + cls_size + 1e-9)
        out_ref[0] = numerical_loss
        out_ref[1] = celoss / numerical_loss
        out_ref[2] = cls_sum / (cls_size + 1e-9)
        out_ref[3] = numer_sum / (n_num + 1e-9)
        out_ref[4] = cls_size
        out_ref[5] = n_cls_i.astype(jnp.float32)
        out_ref[6] = n_num


def kernel(inputs, targets):
    B, C = inputs.shape
    R = 1024
    out = pl.pallas_call(
        _fused_kernel,
        grid=(B // R,),
        in_specs=[
            pl.BlockSpec((R, C), lambda i: (i, 0)),
            pl.BlockSpec((R,), lambda i: (i,)),
        ],
        out_specs=pl.BlockSpec(memory_space=pltpu.SMEM),
        out_shape=jax.ShapeDtypeStruct((8,), jnp.float32),
        scratch_shapes=[
            pltpu.VMEM((B,), jnp.float32),
            pltpu.VMEM((1, 128), jnp.float32),
        ],
    )(inputs, targets)

    return (out[0], out[1], out[2], out[3], out[4], out[5], out[6])
